# Initial kernel scaffold; baseline (speedup 1.0000x reference)
import jax
import jax.numpy as jnp
from jax.experimental import pallas as pl


def kernel(input, target):
    raise NotImplementedError("write your pallas kernel here")


# speedup vs baseline: 10.3006x; 10.3006x over previous
import functools

import jax
import jax.numpy as jnp
from jax import lax
from jax.experimental import pallas as pl
from jax.experimental.pallas import tpu as pltpu
from jax.experimental.pallas import tpu_sc as plsc

_IGNORE_INDEX = 255
_NC = 2
_NS = 16
_NW = _NC * _NS
_L = 16
_CHUNK = 2048


@functools.lru_cache(maxsize=None)
def _make_sc_kernel(B, K, P):
    per_tile = P // _NW
    n_chunks = per_tile // _CHUNK
    n_steps = B * n_chunks
    assert per_tile % _CHUNK == 0 and n_steps % 2 == 0
    groups = _CHUNK // _L

    mesh = plsc.VectorSubcoreMesh(core_axis_name="c", subcore_axis_name="s")

    @functools.partial(
        pl.kernel,
        out_type=jax.ShapeDtypeStruct((_NW * _L,), jnp.float32),
        mesh=mesh,
        compiler_params=pltpu.CompilerParams(
            use_tc_tiling_on_sc=False, needs_layout_passes=False),
        scratch_types=[
            pltpu.VMEM((K, _CHUNK), jnp.float32),
            pltpu.VMEM((K, _CHUNK), jnp.float32),
            pltpu.VMEM((_CHUNK,), jnp.int32),
            pltpu.VMEM((_CHUNK,), jnp.int32),
            pltpu.VMEM((_L,), jnp.float32),
            pltpu.SemaphoreType.DMA,
            pltpu.SemaphoreType.DMA,
        ],
    )
    def grab_and_sum(x_hbm, t_hbm, out_hbm, xbuf0, xbuf1, tbuf0, tbuf1,
                     acc, sem0, sem1):
        cid = lax.axis_index("c")
        sid = lax.axis_index("s")
        wid = sid * _NC + cid
        base = wid * per_tile
        xbufs = (xbuf0, xbuf1)
        tbufs = (tbuf0, tbuf1)
        sems = (sem0, sem1)

        def copies(step, slot):
            b = step // n_chunks
            off = base + (step % n_chunks) * _CHUNK
            return (
                pltpu.make_async_copy(
                    x_hbm.at[b, :, pl.ds(off, _CHUNK)], xbufs[slot],
                    sems[slot],
                ),
                pltpu.make_async_copy(
                    t_hbm.at[b, pl.ds(off, _CHUNK)], tbufs[slot],
                    sems[slot],
                ),
            )

        def start(step, slot):
            for c in copies(step, slot):
                c.start()

        def wait(step, slot):
            for c in copies(step, slot):
                c.wait()

        def consume(slot):
            @pl.loop(0, groups)
            def _(j):
                t16 = tbufs[slot][pl.ds(j * _L, _L)]
                col = lax.iota(jnp.int32, _L) + j * _L
                row = jnp.minimum(t16, K - 1)
                vals = plsc.load_gather(xbufs[slot], [row, col])
                vals = jnp.where(t16 != _IGNORE_INDEX, vals, 0.0)
                acc[...] = acc[...] + vals

        acc[...] = jnp.zeros((_L,), jnp.float32)
        start(0, 0)

        @pl.loop(0, n_steps, step=2)
        def _(i):
            start(i + 1, 1)
            wait(i, 0)
            consume(0)

            @pl.when(i + 2 < n_steps)
            def _():
                start(i + 2, 0)

            wait(i + 1, 1)
            consume(1)

        pltpu.sync_copy(acc, out_hbm.at[pl.ds(wid * _L, _L)])

    return grab_and_sum


def kernel(input, target):
    B, K, N, H, W = input.shape
    P = N * H * W
    x = input.reshape(B, K, P)
    t = target.reshape(B, P).astype(jnp.int32)
    partials = _make_sc_kernel(B, K, P)(x, t)
    return -jnp.sum(partials) / (B * P)
